# baseline (speedup 1.0000x reference)
import functools
import math

import jax
import jax.numpy as jnp
from jax import lax
from jax.experimental import pallas as pl
from jax.experimental.pallas import tpu as pltpu
from jax.experimental.pallas import tpu_sc as plsc

D_MODEL = 64
SCALE = float(math.sqrt(D_MODEL))
BW = 128


@functools.lru_cache(maxsize=None)
def _make_embed(V, D, B, S):
    info = plsc.get_sparse_core_info()
    NC, NS, L = info.num_cores, info.num_subcores, info.num_lanes
    NW = NC * NS
    assert B == BW * NW and D == 64 and L == 16
    n_tok = BW * S
    mesh = plsc.VectorSubcoreMesh(core_axis_name="c", subcore_axis_name="s")

    @functools.partial(
        pl.kernel,
        out_type=jax.ShapeDtypeStruct((S, D // 8, B // BW, 8, BW), jnp.float32),
        mesh=mesh,
        scratch_types=(
            [pltpu.VMEM((n_tok,), jnp.int32)]
            + [pltpu.VMEM((BW,), jnp.int32) for _ in range(8)]
            + [pltpu.VMEM((BW, 2 * D), jnp.float32) for _ in range(4)]
            + [pltpu.VMEM((D // 8, 8, BW), jnp.float32) for _ in range(2)]
            + [pltpu.SemaphoreType.DMA for _ in range(6)]
        ),
        compiler_params=pltpu.CompilerParams(
            use_tc_tiling_on_sc=False, needs_layout_passes=False
        ),
    )
    def k(idx_hbm, tab2_hbm, out5_hbm, idx_all, *scr):
        idxp = scr[0:4]
        p64 = scr[4:8]
        rows = scr[8:12]
        obuf = scr[12:14]
        gsem = scr[14:18]
        ssem = scr[18:20]

        wid = lax.axis_index("s") * NC + lax.axis_index("c")
        pltpu.sync_copy(idx_hbm.at[pl.ds(wid * n_tok, n_tok)], idx_all)

        iota = jnp.arange(L, dtype=jnp.int32)
        iota_s = iota * S

        def prep(l, b):
            vh = jnp.int32(V // 2)

            @plsc.parallel_loop(0, BW, step=L, unroll=4)
            def _(t0):
                tix = iota_s + (t0 * S + l)
                v = plsc.load_gather(idx_all, [tix])
                hi = v >= vh
                idxp[b][pl.ds(t0, L)] = jnp.where(hi, v - vh, v)
                p64[b][pl.ds(t0, L)] = jnp.where(hi, jnp.int32(D), jnp.int32(0))

        def gather_start(b):
            pltpu.async_copy(tab2_hbm.at[idxp[b]], rows[b], gsem[b])

        def gather_wait(b):
            pltpu.make_async_copy(tab2_hbm.at[idxp[b]], rows[b], gsem[b]).wait()

        def trans(b, ob):
            rsels = [iota + g * L for g in range(BW // L)]
            pvs = [p64[b][pl.ds(g * L, L)] for g in range(BW // L)]

            @plsc.parallel_loop(0, D, unroll=8)
            def _(d):
                for g in range(BW // L):
                    v = plsc.load_gather(rows[b], [rsels[g], pvs[g] + d])
                    obuf[ob][d >> 3, d & 7, pl.ds(g * L, L)] = v * SCALE

        def store_start(l, b):
            pltpu.async_copy(obuf[b], out5_hbm.at[l, :, wid], ssem[b])

        def store_wait(l, b):
            pltpu.make_async_copy(obuf[b], out5_hbm.at[l, :, wid], ssem[b]).wait()

        for j in range(3):
            prep(j, j)
            gather_start(j)

        @pl.loop(0, S, step=4)
        def _l0(l0):
            for b in range(4):
                l = l0 + b
                ob = b % 2

                @pl.when(l + 3 < S)
                def _():
                    prep(l + 3, (b + 3) % 4)
                    gather_start((b + 3) % 4)

                gather_wait(b)

                @pl.when(l >= 2)
                def _():
                    store_wait(l, ob)

                trans(b, ob)
                store_start(l, ob)

        store_wait(0, 0)
        store_wait(1, 1)

    return k


@functools.lru_cache(maxsize=None)
def _make_pack(V, D):
    QB = 1000
    G = (V // 2) // QB

    def body(a_ref, b_ref, o_ref):
        o_ref[:, 0:D] = a_ref[...]
        o_ref[:, D : 2 * D] = b_ref[...]

    return pl.pallas_call(
        body,
        grid=(G,),
        in_specs=[
            pl.BlockSpec((QB, D), lambda i: (i, 0)),
            pl.BlockSpec((QB, D), lambda i: (i + G, 0)),
        ],
        out_specs=pl.BlockSpec((QB, 2 * D), lambda i: (i, 0)),
        out_shape=jax.ShapeDtypeStruct((V // 2, 2 * D), jnp.float32),
    )


def kernel(x, table):
    B, S = x.shape
    V, D = table.shape
    idx = x.reshape(-1).astype(jnp.int32)
    tab2 = _make_pack(V, D)(table, table)
    out5 = _make_embed(V, D, B, S)(idx, tab2)
    o = out5.transpose(0, 1, 3, 2, 4)
    o = o.reshape(S, D, B)
    return o.transpose(2, 0, 1)


# speedup vs baseline: 1.1311x; 1.1311x over previous
import functools
import math

import jax
import jax.numpy as jnp
from jax import lax
from jax.experimental import pallas as pl
from jax.experimental.pallas import tpu as pltpu
from jax.experimental.pallas import tpu_sc as plsc

D_MODEL = 64
SCALE = float(math.sqrt(D_MODEL))
BW = 128


@functools.lru_cache(maxsize=None)
def _make_embed(V, D, B, S):
    info = plsc.get_sparse_core_info()
    NC, NS, L = info.num_cores, info.num_subcores, info.num_lanes
    NW = NC * NS
    assert B == BW * NW and D == 64 and L == 16
    n_tok = BW * S
    mesh = plsc.VectorSubcoreMesh(core_axis_name="c", subcore_axis_name="s")

    @functools.partial(
        pl.kernel,
        out_type=jax.ShapeDtypeStruct((S, D // 8, B // BW, 8, BW), jnp.float32),
        mesh=mesh,
        scratch_types=(
            [pltpu.VMEM((n_tok,), jnp.int32)]
            + [pltpu.VMEM((BW,), jnp.int32) for _ in range(8)]
            + [pltpu.VMEM((BW, 2 * D), jnp.float32) for _ in range(4)]
            + [pltpu.VMEM((D // 8, 8, BW), jnp.float32) for _ in range(2)]
            + [pltpu.SemaphoreType.DMA for _ in range(6)]
        ),
        compiler_params=pltpu.CompilerParams(
            use_tc_tiling_on_sc=False, needs_layout_passes=False
        ),
    )
    def k(idx_hbm, tab2_hbm, out5_hbm, idx_all, *scr):
        idxp = scr[0:4]
        p64 = scr[4:8]
        rows = scr[8:12]
        obuf = scr[12:14]
        gsem = scr[14:18]
        ssem = scr[18:20]

        wid = lax.axis_index("s") * NC + lax.axis_index("c")
        pltpu.sync_copy(idx_hbm.at[pl.ds(wid * n_tok, n_tok)], idx_all)

        iota = jnp.arange(L, dtype=jnp.int32)
        iota_s = iota * S

        def prep(l, b):
            @plsc.parallel_loop(0, BW, step=L, unroll=4)
            def _(t0):
                tix = iota_s + (t0 * S + l)
                v = plsc.load_gather(idx_all, [tix])
                idxp[b][pl.ds(t0, L)] = v >> 1
                p64[b][pl.ds(t0, L)] = (v & 1) << 6

        def gather_start(b):
            pltpu.async_copy(tab2_hbm.at[idxp[b]], rows[b], gsem[b])

        def gather_wait(b):
            pltpu.make_async_copy(tab2_hbm.at[idxp[b]], rows[b], gsem[b]).wait()

        def trans(b, ob):
            for g in range(BW // L):
                rsel = iota + g * L
                pv = p64[b][pl.ds(g * L, L)]

                @plsc.parallel_loop(0, D, unroll=16)
                def _(d):
                    v = plsc.load_gather(rows[b], [rsel, pv + d])
                    obuf[ob][d >> 3, d & 7, pl.ds(g * L, L)] = v * SCALE

        def store_start(l, b):
            pltpu.async_copy(obuf[b], out5_hbm.at[l, :, wid], ssem[b])

        def store_wait(l, b):
            pltpu.make_async_copy(obuf[b], out5_hbm.at[l, :, wid], ssem[b]).wait()

        for j in range(3):
            prep(j, j)
            gather_start(j)

        @pl.loop(0, S, step=4)
        def _l0(l0):
            for b in range(4):
                l = l0 + b
                ob = b % 2

                @pl.when(l + 3 < S)
                def _():
                    prep(l + 3, (b + 3) % 4)
                    gather_start((b + 3) % 4)

                gather_wait(b)

                @pl.when(l >= 2)
                def _():
                    store_wait(l, ob)

                trans(b, ob)
                store_start(l, ob)

        store_wait(0, 0)
        store_wait(1, 1)

    return k


@functools.lru_cache(maxsize=None)
def _make_pack(V, D):
    QB = 1000
    G = (V // 2) // QB

    def body(a_ref, b_ref, o_ref):
        o_ref[:, 0:D] = a_ref[...]
        o_ref[:, D : 2 * D] = b_ref[...]

    return pl.pallas_call(
        body,
        grid=(G,),
        in_specs=[
            pl.BlockSpec((QB, D), lambda i: (i, 0)),
            pl.BlockSpec((QB, D), lambda i: (i + G, 0)),
        ],
        out_specs=pl.BlockSpec((QB, 2 * D), lambda i: (i, 0)),
        out_shape=jax.ShapeDtypeStruct((V // 2, 2 * D), jnp.float32),
    )


def kernel(x, table):
    B, S = x.shape
    V, D = table.shape
    idx = x.reshape(-1).astype(jnp.int32)
    tab2 = table.reshape(V // 2, 2 * D)
    out5 = _make_embed(V, D, B, S)(idx, tab2)
    o = out5.transpose(0, 1, 3, 2, 4)
    o = o.reshape(S, D, B)
    return o.transpose(2, 0, 1)


# speedup vs baseline: 1.1422x; 1.0098x over previous
import functools
import math

import jax
import jax.numpy as jnp
from jax import lax
from jax.experimental import pallas as pl
from jax.experimental.pallas import tpu as pltpu
from jax.experimental.pallas import tpu_sc as plsc

D_MODEL = 64
SCALE = float(math.sqrt(D_MODEL))
BW = 128


@functools.lru_cache(maxsize=None)
def _make_embed(V, D, B, S):
    info = plsc.get_sparse_core_info()
    NC, NS, L = info.num_cores, info.num_subcores, info.num_lanes
    NW = NC * NS
    assert B == BW * NW and D == 64 and L == 16
    n_tok = BW * S
    mesh = plsc.VectorSubcoreMesh(core_axis_name="c", subcore_axis_name="s")

    @functools.partial(
        pl.kernel,
        out_type=jax.ShapeDtypeStruct((S, D // 8, B // BW, 8, BW), jnp.float32),
        mesh=mesh,
        scratch_types=(
            [pltpu.VMEM((n_tok,), jnp.int32)]
            + [pltpu.VMEM((BW,), jnp.int32) for _ in range(8)]
            + [pltpu.VMEM((BW, 2 * D), jnp.float32) for _ in range(4)]
            + [pltpu.VMEM((D, BW), jnp.float32) for _ in range(2)]
            + [pltpu.SemaphoreType.DMA for _ in range(6)]
        ),
        compiler_params=pltpu.CompilerParams(
            use_tc_tiling_on_sc=False, needs_layout_passes=False
        ),
    )
    def k(idx_hbm, tab2_hbm, out5_hbm, idx_all, *scr):
        idxp = scr[0:4]
        p64 = scr[4:8]
        rows = scr[8:12]
        obuf = scr[12:14]
        gsem = scr[14:18]
        ssem = scr[18:20]

        wid = lax.axis_index("s") * NC + lax.axis_index("c")
        pltpu.sync_copy(idx_hbm.at[pl.ds(wid * n_tok, n_tok)], idx_all)

        iota = jnp.arange(L, dtype=jnp.int32)
        iota_s = iota * S

        def prep(l, b):
            @plsc.parallel_loop(0, BW, step=L, unroll=4)
            def _(t0):
                tix = iota_s + (t0 * S + l)
                v = plsc.load_gather(idx_all, [tix])
                idxp[b][pl.ds(t0, L)] = v >> 1
                p64[b][pl.ds(t0, L)] = (v & 1) << 6

        def gather_start(b):
            pltpu.async_copy(tab2_hbm.at[idxp[b]], rows[b], gsem[b])

        def gather_wait(b):
            pltpu.make_async_copy(tab2_hbm.at[idxp[b]], rows[b], gsem[b]).wait()

        def trans(b, ob):
            rsels = [iota + g * L for g in range(BW // L)]
            pvs = [p64[b][pl.ds(g * L, L)] for g in range(BW // L)]

            @plsc.parallel_loop(0, D, unroll=8)
            def _(d):
                for g in range(BW // L):
                    v = plsc.load_gather(rows[b], [rsels[g], pvs[g] + d])
                    obuf[ob][d, pl.ds(g * L, L)] = v * SCALE

        def store_start(l, b):
            for di in range(D // 8):
                pltpu.async_copy(
                    obuf[b].at[pl.ds(di * 8, 8)], out5_hbm.at[l, di, wid], ssem[b]
                )

        def store_wait(l, b):
            for di in range(D // 8):
                pltpu.make_async_copy(
                    obuf[b].at[pl.ds(di * 8, 8)], out5_hbm.at[l, di, wid], ssem[b]
                ).wait()

        for j in range(3):
            prep(j, j)
            gather_start(j)

        @pl.loop(0, S, step=4)
        def _l0(l0):
            for b in range(4):
                l = l0 + b
                ob = b % 2

                @pl.when(l + 3 < S)
                def _():
                    prep(l + 3, (b + 3) % 4)
                    gather_start((b + 3) % 4)

                gather_wait(b)

                @pl.when(l >= 2)
                def _():
                    store_wait(l, ob)

                trans(b, ob)
                store_start(l, ob)

        store_wait(0, 0)
        store_wait(1, 1)

    return k


@functools.lru_cache(maxsize=None)
def _make_pack(V, D):
    QB = 1000
    G = (V // 2) // QB

    def body(a_ref, b_ref, o_ref):
        o_ref[:, 0:D] = a_ref[...]
        o_ref[:, D : 2 * D] = b_ref[...]

    return pl.pallas_call(
        body,
        grid=(G,),
        in_specs=[
            pl.BlockSpec((QB, D), lambda i: (i, 0)),
            pl.BlockSpec((QB, D), lambda i: (i + G, 0)),
        ],
        out_specs=pl.BlockSpec((QB, 2 * D), lambda i: (i, 0)),
        out_shape=jax.ShapeDtypeStruct((V // 2, 2 * D), jnp.float32),
    )


def kernel(x, table):
    B, S = x.shape
    V, D = table.shape
    idx = x.reshape(-1).astype(jnp.int32)
    tab2 = table.reshape(V // 2, 2 * D)
    out5 = _make_embed(V, D, B, S)(idx, tab2)
    o = out5.transpose(0, 1, 3, 2, 4)
    o = o.reshape(S, D, B)
    return o.transpose(2, 0, 1)


# speedup vs baseline: 1.1747x; 1.0284x over previous
import functools
import math

import jax
import jax.numpy as jnp
from jax import lax
from jax.experimental import pallas as pl
from jax.experimental.pallas import tpu as pltpu
from jax.experimental.pallas import tpu_sc as plsc

D_MODEL = 64
SCALE = math.sqrt(D_MODEL)
CHUNK = 320
NBUF = 4


@functools.lru_cache(maxsize=None)
def _make_gather(V, D, B_total):
    info = plsc.get_sparse_core_info()
    NC, NS, L = info.num_cores, info.num_subcores, info.num_lanes
    NW = NC * NS
    assert B_total % NW == 0
    n_per_w = B_total // NW
    assert n_per_w % (CHUNK * NBUF) == 0
    n_chunks = n_per_w // CHUNK
    mesh = plsc.VectorSubcoreMesh(core_axis_name="c", subcore_axis_name="s")

    @functools.partial(
        pl.kernel,
        out_type=jax.ShapeDtypeStruct((B_total, D), jnp.float32),
        mesh=mesh,
        scratch_types=(
            [pltpu.VMEM((n_per_w,), jnp.int32)]
            + [pltpu.VMEM((CHUNK, D), jnp.float32) for _ in range(NBUF)]
            + [pltpu.SemaphoreType.DMA for _ in range(2 * NBUF)]
        ),
        compiler_params=pltpu.CompilerParams(use_tc_tiling_on_sc=False),
    )
    def k(idx_hbm, table_hbm, out_hbm, idx_all, *scratch):
        rows = scratch[:NBUF]
        gsem = scratch[NBUF : 2 * NBUF]
        ssem = scratch[2 * NBUF : 3 * NBUF]
        wid = lax.axis_index("s") * NC + lax.axis_index("c")
        base = wid * n_per_w

        pltpu.sync_copy(idx_hbm.at[pl.ds(base, n_per_w)], idx_all)

        def gather_start(c, b):
            pltpu.async_copy(
                table_hbm.at[idx_all.at[pl.ds(c * CHUNK, CHUNK)]], rows[b], gsem[b]
            )

        def gather_wait(c, b):
            pltpu.make_async_copy(
                table_hbm.at[idx_all.at[pl.ds(c * CHUNK, CHUNK)]], rows[b], gsem[b]
            ).wait()

        def store_start(c, b):
            pltpu.async_copy(rows[b], out_hbm.at[pl.ds(base + c * CHUNK, CHUNK)], ssem[b])

        def store_wait(c, b):
            pltpu.make_async_copy(
                rows[b], out_hbm.at[pl.ds(base + c * CHUNK, CHUNK)], ssem[b]
            ).wait()

        for b in range(NBUF):
            gather_start(b, b)

        @pl.loop(0, n_chunks, step=NBUF)
        def _step(g0):
            for b in range(NBUF):
                c = g0 + b
                gather_wait(c, b)

                @pl.loop(0, CHUNK, unroll=8)
                def _scale(i):
                    for j in range(D // L):
                        sl = pl.ds(j * L, L)
                        rows[b][i, sl] = rows[b][i, sl] * SCALE

                store_start(c, b)
            for b in range(NBUF):
                c = g0 + b
                store_wait(c, b)
                n = c + NBUF

                @pl.when(n < n_chunks)
                def _():
                    gather_start(n, b)

    return k


def kernel(x, table):
    B, S = x.shape
    V, D = table.shape
    idx = x.reshape(-1).astype(jnp.int32)
    out = _make_gather(V, D, B * S)(idx, table)
    return out.reshape(B, S, D)
